# baseline (speedup 1.0000x reference)
import functools

import jax
import jax.numpy as jnp
from jax import lax
from jax.experimental import pallas as pl
from jax.experimental.pallas import tpu as pltpu
from jax.experimental.pallas import tpu_sc as plsc

B = 8
LQ = 32
LK = 2048
D = 64
MAX_REL = 128
NPAIR = B * LQ
NW = 32
PAIRS_PER_W = NPAIR // NW
CHUNK = 128
NCHUNK = LK // CHUNK

_mesh = plsc.VectorSubcoreMesh(core_axis_name="c", subcore_axis_name="s")


@functools.partial(
    pl.kernel,
    mesh=_mesh,
    compiler_params=pltpu.CompilerParams(use_tc_tiling_on_sc=False),
    out_type=jax.ShapeDtypeStruct((NPAIR * LK, D), jnp.float32),
    scratch_types=[
        pltpu.VMEM((B * LK,), jnp.int32),
        pltpu.VMEM((NPAIR + 16,), jnp.int32),
        pltpu.VMEM((CHUNK,), jnp.int32),
        pltpu.VMEM((CHUNK, D), jnp.float32),
        pltpu.SemaphoreType.DMA,
    ],
)
def _sc_gather(pq_hbm, pk_hbm, table_hbm, out_hbm, pk_v, pq_v, idx_v, rows_v, sem):
    wid = lax.axis_index("s") * 2 + lax.axis_index("c")
    pltpu.sync_copy(pk_hbm, pk_v)
    pltpu.sync_copy(pq_hbm, pq_v)
    def pair_body(p, carry):
        pair = wid * PAIRS_PER_W + p
        bq = pair // LQ
        pq_scalar = pq_v[pl.ds(pair, 16)][0]
        pq_splat = jnp.full((16,), pq_scalar, jnp.int32)
        pk_base = bq * LK

        def chunk_body(c, carry2):
            def vec_body(v, carry3):
                pk16 = pk_v[pl.ds(pk_base + c * CHUNK + v * 16, 16)]
                d = jnp.clip(pk16 - pq_splat, -MAX_REL, MAX_REL) + MAX_REL
                idx_v[pl.ds(v * 16, 16)] = d
                return carry3

            lax.fori_loop(0, CHUNK // 16, vec_body, 0, unroll=True)
            pltpu.async_copy(table_hbm.at[idx_v], rows_v, sem).wait()
            pltpu.sync_copy(
                rows_v, out_hbm.at[pl.ds(pair * LK + c * CHUNK, CHUNK)]
            )
            return carry2

        lax.fori_loop(0, NCHUNK, chunk_body, 0)
        return carry

    lax.fori_loop(0, PAIRS_PER_W, pair_body, 0)


def kernel(position_q, position_k, embeddings_table):
    pq = position_q.astype(jnp.int32).reshape(NPAIR)
    pq = jnp.pad(pq, (0, 16))
    pk = position_k.astype(jnp.int32).reshape(B * LK)
    out = _sc_gather(pq, pk, embeddings_table)
    return out.reshape(B, LQ, LK, D)


# speedup vs baseline: 1.0145x; 1.0145x over previous
import functools

import jax
import jax.numpy as jnp
from jax import lax
from jax.experimental import pallas as pl
from jax.experimental.pallas import tpu as pltpu
from jax.experimental.pallas import tpu_sc as plsc

B = 8
LQ = 32
LK = 2048
D = 64
MAX_REL = 128
NPAIR = B * LQ
NW = 32
PAIRS_PER_W = NPAIR // NW
Q = 256
S = (PAIRS_PER_W * LK) // Q
STEPS_PER_PAIR = LK // Q
NSLOT = 4

_mesh = plsc.VectorSubcoreMesh(core_axis_name="c", subcore_axis_name="s")


@functools.partial(
    pl.kernel,
    mesh=_mesh,
    compiler_params=pltpu.CompilerParams(use_tc_tiling_on_sc=False),
    out_type=jax.ShapeDtypeStruct((NPAIR * LK // 128, 128, D), jnp.float32),
    scratch_types=[
        pltpu.VMEM((LK,), jnp.int32),
        pltpu.VMEM((NPAIR + 16,), jnp.int32),
        pltpu.VMEM((NSLOT, 2, 128), jnp.int32),
        pltpu.VMEM((NSLOT, 2, 128, D), jnp.float32),
        pltpu.SemaphoreType.DMA((NSLOT,)),
        pltpu.SemaphoreType.DMA((NSLOT,)),
    ],
)
def _sc_gather(pq_hbm, pk_hbm, table_hbm, out_hbm, pk_v, pq_v, idx_v, rows_v,
               gsem, ssem):
    wid = lax.axis_index("s") * 2 + lax.axis_index("c")
    bq = wid // (LQ // PAIRS_PER_W)
    pltpu.sync_copy(pk_hbm.at[pl.ds(bq * LK, LK)], pk_v)
    pltpu.sync_copy(pq_hbm, pq_v)
    out_base = wid * (S * (Q // 128))

    def compute_idx(t, k):
        pq_scalar = pq_v[pl.ds(wid * PAIRS_PER_W + t // STEPS_PER_PAIR, 16)][0]
        pq_splat = jnp.full((16,), pq_scalar, jnp.int32)
        jbase = (t % STEPS_PER_PAIR) * Q
        for v in range(Q // 16):
            pk16 = pk_v[pl.ds(jbase + v * 16, 16)]
            d = jnp.clip(pk16 - pq_splat, -MAX_REL, MAX_REL) + MAX_REL
            idx_v[k, v // 8, pl.ds((v % 8) * 16, 16)] = d

    def fire_gather(t, k):
        compute_idx(t, k)
        for h in range(2):
            pltpu.async_copy(
                table_hbm.at[idx_v.at[k, h]], rows_v.at[k, h], gsem.at[k]
            )

    def wait_gather(k):
        for h in range(2):
            pltpu.make_async_copy(
                table_hbm.at[idx_v.at[k, h]], rows_v.at[k, h], gsem.at[k]
            ).wait()

    def fire_scatter(t, k):
        pltpu.async_copy(
            rows_v.at[k], out_hbm.at[pl.ds(out_base + t * 2, 2)], ssem.at[k]
        )

    def wait_scatter(k):
        pltpu.make_async_copy(
            rows_v.at[k], out_hbm.at[pl.ds(0, 2)], ssem.at[k]
        ).wait()

    def iter_body(i, k, do_swait, do_b):
        wait_gather(k)
        fire_scatter(i, k)
        if do_b:
            kb = (k + 2) % NSLOT
            if do_swait:
                wait_scatter(kb)
            fire_gather(i + 2, kb)

    fire_gather(0, 0)
    fire_gather(1, 1)
    for k in range(NSLOT):
        iter_body(k, k, do_swait=(k >= 2), do_b=True)

    def outer(o, carry):
        for k in range(NSLOT):
            iter_body(o * NSLOT + k, k, True, True)
        return carry

    lax.fori_loop(1, S // NSLOT - 1, outer, 0)

    for k in range(NSLOT):
        i = S - NSLOT + k
        iter_body(i, k, do_swait=True, do_b=(i < S - 2))
    for k in range(NSLOT):
        wait_scatter(k)


def kernel(position_q, position_k, embeddings_table):
    pq = position_q.astype(jnp.int32).reshape(NPAIR)
    pq = jnp.pad(pq, (0, 16))
    pk = position_k.astype(jnp.int32).reshape(B * LK)
    out = _sc_gather(pq, pk, embeddings_table)
    return out.reshape(B, LQ, LK, D)


# speedup vs baseline: 3.3080x; 3.2607x over previous
import functools

import jax
import jax.numpy as jnp
from jax import lax
from jax.experimental import pallas as pl
from jax.experimental.pallas import tpu as pltpu
from jax.experimental.pallas import tpu_sc as plsc

B = 8
LQ = 32
LK = 2048
D = 64
MAX_REL = 128
NROW = 2 * MAX_REL + 1
NPAIR = B * LQ
NW = 32
PAIRS_PER_W = NPAIR // NW
Q = 256
S = (PAIRS_PER_W * LK) // Q
STEPS_PER_PAIR = LK // Q
NSLOT = 2

_mesh = plsc.VectorSubcoreMesh(core_axis_name="c", subcore_axis_name="s")


@functools.partial(
    pl.kernel,
    mesh=_mesh,
    compiler_params=pltpu.CompilerParams(
        use_tc_tiling_on_sc=False, needs_layout_passes=False
    ),
    out_type=jax.ShapeDtypeStruct((NPAIR * LK * D,), jnp.float32),
    scratch_types=[
        pltpu.VMEM((NROW * D,), jnp.float32),
        pltpu.VMEM((LK,), jnp.int32),
        pltpu.VMEM((NPAIR + 16,), jnp.int32),
        pltpu.VMEM((NSLOT * Q * D,), jnp.float32),
        pltpu.SemaphoreType.DMA,
        pltpu.SemaphoreType.DMA,
    ],
)
def _sc_gather(pq_hbm, pk_hbm, table_hbm, out_hbm, tab_v, pk_v, pq_v, rows_v,
               ssem0, ssem1):
    wid = lax.axis_index("s") * 2 + lax.axis_index("c")
    bq = wid // (LQ // PAIRS_PER_W)
    pltpu.sync_copy(table_hbm, tab_v)
    pltpu.sync_copy(pk_hbm.at[pl.ds(bq * LK, LK)], pk_v)
    pltpu.sync_copy(pq_hbm, pq_v)
    row_base = wid * (S * Q)
    lane64 = lax.iota(jnp.int32, 16) * D

    def build_step(t, k):
        pq_scalar = pq_v[pl.ds(wid * PAIRS_PER_W + t // STEPS_PER_PAIR, 16)][0]
        pq_splat = jnp.full((16,), pq_scalar, jnp.int32)
        jbase = (t % STEPS_PER_PAIR) * Q
        lane64k = lane64 + k * (Q * D)
        for g in range(Q // 16):
            pk16 = pk_v[pl.ds(jbase + g * 16, 16)]
            flat = (jnp.clip(pk16 - pq_splat, -MAX_REL, MAX_REL) + MAX_REL) * D
            dst_base = lane64k + g * 16 * D

            def col_body(cb, carry):
                c0 = cb * 8
                for dc in range(8):
                    vals = plsc.load_gather(tab_v, [flat + (c0 + dc)])
                    plsc.store_scatter(rows_v, [dst_base + (c0 + dc)], vals)
                return carry

            lax.fori_loop(0, D // 8, col_body, 0)

    def fire_scatter(t, k, sem):
        pltpu.async_copy(
            rows_v.at[pl.ds(k * (Q * D), Q * D)],
            out_hbm.at[pl.ds((row_base + t * Q) * D, Q * D)],
            sem,
        )

    def wait_scatter(k, sem):
        pltpu.make_async_copy(
            rows_v.at[pl.ds(k * (Q * D), Q * D)],
            out_hbm.at[pl.ds(0, Q * D)],
            sem,
        ).wait()

    build_step(0, 0)
    fire_scatter(0, 0, ssem0)
    build_step(1, 1)
    fire_scatter(1, 1, ssem1)

    def outer(o, carry):
        s0 = o * 2
        wait_scatter(0, ssem0)
        build_step(s0, 0)
        fire_scatter(s0, 0, ssem0)
        wait_scatter(1, ssem1)
        build_step(s0 + 1, 1)
        fire_scatter(s0 + 1, 1, ssem1)
        return carry

    lax.fori_loop(1, S // 2, outer, 0)
    wait_scatter(0, ssem0)
    wait_scatter(1, ssem1)


def kernel(position_q, position_k, embeddings_table):
    pq = position_q.astype(jnp.int32).reshape(NPAIR)
    pq = jnp.pad(pq, (0, 16))
    pk = position_k.astype(jnp.int32).reshape(B * LK)
    tab = embeddings_table.reshape(NROW * D)
    out = _sc_gather(pq, pk, tab)
    return out.reshape(B, LQ, LK, D)


# speedup vs baseline: 6.8163x; 2.0606x over previous
import functools

import jax
import jax.numpy as jnp
from jax import lax
from jax.experimental import pallas as pl
from jax.experimental.pallas import tpu as pltpu
from jax.experimental.pallas import tpu_sc as plsc

B = 8
LQ = 32
LK = 2048
D = 64
MAX_REL = 128
NROW = 2 * MAX_REL + 1
NPAIR = B * LQ
NW = 32
PAIRS_PER_W = NPAIR // NW
Q = 256
S = (PAIRS_PER_W * LK) // Q
STEPS_PER_PAIR = LK // Q
NSLOT = 2

_mesh = plsc.VectorSubcoreMesh(core_axis_name="c", subcore_axis_name="s")


@functools.partial(
    pl.kernel,
    mesh=_mesh,
    compiler_params=pltpu.CompilerParams(
        use_tc_tiling_on_sc=False, needs_layout_passes=False
    ),
    out_type=jax.ShapeDtypeStruct((NPAIR * LK * D,), jnp.float32),
    scratch_types=[
        pltpu.VMEM((NROW * D,), jnp.float32),
        pltpu.VMEM((LK,), jnp.int32),
        pltpu.VMEM((NPAIR + 16,), jnp.int32),
        pltpu.VMEM((32,), jnp.int32),
        pltpu.VMEM((NSLOT * Q * D,), jnp.float32),
        pltpu.SemaphoreType.DMA,
        pltpu.SemaphoreType.DMA,
    ],
)
def _sc_gather(pq_hbm, pk_hbm, table_hbm, out_hbm, tab_v, pk_v, pq_v, idx_v,
               rows_v, ssem0, ssem1):
    wid = lax.axis_index("s") * 2 + lax.axis_index("c")
    bq = wid // (LQ // PAIRS_PER_W)
    pltpu.sync_copy(table_hbm, tab_v)
    pltpu.sync_copy(pk_hbm.at[pl.ds(bq * LK, LK)], pk_v)
    pltpu.sync_copy(pq_hbm, pq_v)
    row_base = wid * (S * Q)

    def build_step(t, k):
        pq_scalar = pq_v[pl.ds(wid * PAIRS_PER_W + t // STEPS_PER_PAIR, 16)][0]
        pq_splat = jnp.full((16,), pq_scalar, jnp.int32)
        jbase = (t % STEPS_PER_PAIR) * Q

        def grp_body(g, carry):
            pk16 = pk_v[pl.ds(jbase + g * 16, 16)]
            flat = (jnp.clip(pk16 - pq_splat, -MAX_REL, MAX_REL) + MAX_REL) * D
            idx_v[pl.ds(0, 16)] = flat
            for r in range(16):
                src = idx_v[pl.ds(r, 16)][0]
                dst = k * (Q * D) + g * (16 * D) + r * D
                for c4 in range(D // 16):
                    rows_v[pl.ds(dst + c4 * 16, 16)] = (
                        tab_v[pl.ds(src + c4 * 16, 16)]
                    )
            return carry

        lax.fori_loop(0, Q // 16, grp_body, 0)

    def fire_scatter(t, k, sem):
        pltpu.async_copy(
            rows_v.at[pl.ds(k * (Q * D), Q * D)],
            out_hbm.at[pl.ds((row_base + t * Q) * D, Q * D)],
            sem,
        )

    def wait_scatter(k, sem):
        pltpu.make_async_copy(
            rows_v.at[pl.ds(k * (Q * D), Q * D)],
            out_hbm.at[pl.ds(0, Q * D)],
            sem,
        ).wait()

    build_step(0, 0)
    fire_scatter(0, 0, ssem0)
    build_step(1, 1)
    fire_scatter(1, 1, ssem1)

    def outer(o, carry):
        s0 = o * 2
        wait_scatter(0, ssem0)
        build_step(s0, 0)
        fire_scatter(s0, 0, ssem0)
        wait_scatter(1, ssem1)
        build_step(s0 + 1, 1)
        fire_scatter(s0 + 1, 1, ssem1)
        return carry

    lax.fori_loop(1, S // 2, outer, 0)
    wait_scatter(0, ssem0)
    wait_scatter(1, ssem1)


def kernel(position_q, position_k, embeddings_table):
    pq = position_q.astype(jnp.int32).reshape(NPAIR)
    pq = jnp.pad(pq, (0, 16))
    pk = position_k.astype(jnp.int32).reshape(B * LK)
    tab = embeddings_table.reshape(NROW * D)
    out = _sc_gather(pq, pk, tab)
    return out.reshape(B, LQ, LK, D)


# speedup vs baseline: 9.1169x; 1.3375x over previous
import functools

import jax
import jax.numpy as jnp
from jax import lax
from jax.experimental import pallas as pl
from jax.experimental.pallas import tpu as pltpu
from jax.experimental.pallas import tpu_sc as plsc

B = 8
LQ = 32
LK = 2048
D = 64
MAX_REL = 128
NROW = 2 * MAX_REL + 1
NPAIR = B * LQ
NW = 32
PAIRS_PER_W = NPAIR // NW
Q = 256
S = (PAIRS_PER_W * LK) // Q
STEPS_PER_PAIR = LK // Q
NSLOT = 2

_mesh = plsc.VectorSubcoreMesh(core_axis_name="c", subcore_axis_name="s")


@functools.partial(
    pl.kernel,
    mesh=_mesh,
    compiler_params=pltpu.CompilerParams(
        use_tc_tiling_on_sc=False, needs_layout_passes=False
    ),
    out_type=jax.ShapeDtypeStruct((NPAIR * LK * D,), jnp.float32),
    scratch_types=[
        pltpu.VMEM((NROW * D,), jnp.float32),
        pltpu.VMEM((LK,), jnp.int32),
        pltpu.VMEM((NPAIR + 16,), jnp.int32),
        pltpu.VMEM((32,), jnp.int32),
        pltpu.VMEM((NSLOT * Q * D,), jnp.float32),
        pltpu.SemaphoreType.DMA,
        pltpu.SemaphoreType.DMA,
    ],
)
def _sc_gather(pq_hbm, pk_hbm, table_hbm, out_hbm, tab_v, pk_v, pq_v, idx_v,
               rows_v, ssem0, ssem1):
    wid = lax.axis_index("s") * 2 + lax.axis_index("c")
    bq = wid // (LQ // PAIRS_PER_W)
    pltpu.sync_copy(table_hbm, tab_v)
    pltpu.sync_copy(pk_hbm.at[pl.ds(bq * LK, LK)], pk_v)
    pltpu.sync_copy(pq_hbm, pq_v)
    row_base = wid * (S * Q)
    lane = lax.iota(jnp.int32, 16)
    cvecs = [lane + c4 * 16 for c4 in range(D // 16)]
    _dnums = lax.GatherDimensionNumbers(
        offset_dims=(), collapsed_slice_dims=(0,), start_index_map=(0,)
    )

    def _splat_lane(vec, r):
        return lax.gather(
            vec, jnp.full((16, 1), r, jnp.int32), _dnums, (1,),
            mode=lax.GatherScatterMode.PROMISE_IN_BOUNDS,
        )

    def build_step(t, k):
        pq_scalar = pq_v[pl.ds(wid * PAIRS_PER_W + t // STEPS_PER_PAIR, 16)][0]
        pq_splat = jnp.full((16,), pq_scalar, jnp.int32)
        jbase = (t % STEPS_PER_PAIR) * Q

        def grp_body(g, carry):
            pk16 = pk_v[pl.ds(jbase + g * 16, 16)]
            flat = (jnp.clip(pk16 - pq_splat, -MAX_REL, MAX_REL) + MAX_REL) * D
            for r in range(16):
                rowbase = _splat_lane(flat, r)
                dst = k * (Q * D) + g * (16 * D) + r * D
                for c4 in range(D // 16):
                    vals = plsc.load_gather(tab_v, [rowbase + cvecs[c4]])
                    rows_v[pl.ds(dst + c4 * 16, 16)] = vals
            return carry

        lax.fori_loop(0, Q // 16, grp_body, 0)

    def fire_scatter(t, k, sem):
        pltpu.async_copy(
            rows_v.at[pl.ds(k * (Q * D), Q * D)],
            out_hbm.at[pl.ds((row_base + t * Q) * D, Q * D)],
            sem,
        )

    def wait_scatter(k, sem):
        pltpu.make_async_copy(
            rows_v.at[pl.ds(k * (Q * D), Q * D)],
            out_hbm.at[pl.ds(0, Q * D)],
            sem,
        ).wait()

    build_step(0, 0)
    fire_scatter(0, 0, ssem0)
    build_step(1, 1)
    fire_scatter(1, 1, ssem1)

    def outer(o, carry):
        s0 = o * 2
        wait_scatter(0, ssem0)
        build_step(s0, 0)
        fire_scatter(s0, 0, ssem0)
        wait_scatter(1, ssem1)
        build_step(s0 + 1, 1)
        fire_scatter(s0 + 1, 1, ssem1)
        return carry

    lax.fori_loop(1, S // 2, outer, 0)
    wait_scatter(0, ssem0)
    wait_scatter(1, ssem1)


def kernel(position_q, position_k, embeddings_table):
    pq = position_q.astype(jnp.int32).reshape(NPAIR)
    pq = jnp.pad(pq, (0, 16))
    pk = position_k.astype(jnp.int32).reshape(B * LK)
    tab = embeddings_table.reshape(NROW * D)
    out = _sc_gather(pq, pk, tab)
    return out.reshape(B, LQ, LK, D)


# speedup vs baseline: 12.7030x; 1.3934x over previous
import functools

import jax
import jax.numpy as jnp
from jax import lax
from jax.experimental import pallas as pl
from jax.experimental.pallas import tpu as pltpu
from jax.experimental.pallas import tpu_sc as plsc

B = 8
LQ = 32
LK = 2048
D = 64
MAX_REL = 128
NROW = 2 * MAX_REL + 1
NPAIR = B * LQ
NW = 32
PAIRS_PER_W = NPAIR // NW
Q = 256
S = (PAIRS_PER_W * LK) // Q
STEPS_PER_PAIR = LK // Q
NSLOT = 2

_mesh = plsc.VectorSubcoreMesh(core_axis_name="c", subcore_axis_name="s")


@functools.partial(
    pl.kernel,
    mesh=_mesh,
    compiler_params=pltpu.CompilerParams(
        use_tc_tiling_on_sc=False, needs_layout_passes=False
    ),
    out_type=jax.ShapeDtypeStruct((NPAIR * LK * D,), jnp.float32),
    scratch_types=[
        pltpu.VMEM((NROW * D,), jnp.float32),
        pltpu.VMEM((LK,), jnp.int32),
        pltpu.VMEM((NPAIR + 16,), jnp.int32),
        pltpu.VMEM((32,), jnp.int32),
        pltpu.VMEM((NSLOT * Q * D,), jnp.float32),
        pltpu.SemaphoreType.DMA,
        pltpu.SemaphoreType.DMA,
    ],
)
def _sc_gather(pq_hbm, pk_hbm, table_hbm, out_hbm, tab_v, pk_v, pq_v, idx_v,
               rows_v, ssem0, ssem1):
    wid = lax.axis_index("s") * 2 + lax.axis_index("c")
    bq = wid // (LQ // PAIRS_PER_W)
    pltpu.sync_copy(table_hbm, tab_v)
    pltpu.sync_copy(pk_hbm.at[pl.ds(bq * LK, LK)], pk_v)
    pltpu.sync_copy(pq_hbm, pq_v)
    row_base = wid * (S * Q)
    lane = lax.iota(jnp.int32, 16)
    cvecs = [lane + c4 * 16 for c4 in range(D // 16)]
    _dnums = lax.GatherDimensionNumbers(
        offset_dims=(), collapsed_slice_dims=(0,), start_index_map=(0,)
    )

    def _splat_lane(vec, r):
        return lax.gather(
            vec, jnp.full((16, 1), r, jnp.int32), _dnums, (1,),
            mode=lax.GatherScatterMode.PROMISE_IN_BOUNDS,
        )

    def _splat_lane_dyn(vec, r):
        return lax.gather(
            vec, jnp.broadcast_to(r, (16, 1)).astype(jnp.int32), _dnums, (1,),
            mode=lax.GatherScatterMode.PROMISE_IN_BOUNDS,
        )

    def build_step(t, k):
        pq_scalar = pq_v[pl.ds(wid * PAIRS_PER_W + t // STEPS_PER_PAIR, 16)][0]
        pq_splat = jnp.full((16,), pq_scalar, jnp.int32)
        jbase = (t % STEPS_PER_PAIR) * Q

        def grp_body(g, carry):
            pk16 = pk_v[pl.ds(jbase + g * 16, 16)]
            flat = (jnp.clip(pk16 - pq_splat, -MAX_REL, MAX_REL) + MAX_REL) * D
            dst_g = k * (Q * D) + g * (16 * D)

            @plsc.parallel_loop(0, 16, 1, unroll=8)
            def _rows(r):
                rowbase = _splat_lane_dyn(flat, r)
                dst = dst_g + r * D
                for c4 in range(D // 16):
                    vals = plsc.load_gather(tab_v, [rowbase + cvecs[c4]])
                    rows_v[pl.ds(dst + c4 * 16, 16)] = vals

            return carry

        lax.fori_loop(0, Q // 16, grp_body, 0)

    def fire_scatter(t, k, sem):
        pltpu.async_copy(
            rows_v.at[pl.ds(k * (Q * D), Q * D)],
            out_hbm.at[pl.ds((row_base + t * Q) * D, Q * D)],
            sem,
        )

    def wait_scatter(k, sem):
        pltpu.make_async_copy(
            rows_v.at[pl.ds(k * (Q * D), Q * D)],
            out_hbm.at[pl.ds(0, Q * D)],
            sem,
        ).wait()

    build_step(0, 0)
    fire_scatter(0, 0, ssem0)
    build_step(1, 1)
    fire_scatter(1, 1, ssem1)

    def outer(o, carry):
        s0 = o * 2
        wait_scatter(0, ssem0)
        build_step(s0, 0)
        fire_scatter(s0, 0, ssem0)
        wait_scatter(1, ssem1)
        build_step(s0 + 1, 1)
        fire_scatter(s0 + 1, 1, ssem1)
        return carry

    lax.fori_loop(1, S // 2, outer, 0)
    wait_scatter(0, ssem0)
    wait_scatter(1, ssem1)


def kernel(position_q, position_k, embeddings_table):
    pq = position_q.astype(jnp.int32).reshape(NPAIR)
    pq = jnp.pad(pq, (0, 16))
    pk = position_k.astype(jnp.int32).reshape(B * LK)
    tab = embeddings_table.reshape(NROW * D)
    out = _sc_gather(pq, pk, tab)
    return out.reshape(B, LQ, LK, D)


# speedup vs baseline: 17.7010x; 1.3934x over previous
import functools

import jax
import jax.numpy as jnp
from jax import lax
from jax.experimental import pallas as pl
from jax.experimental.pallas import tpu as pltpu
from jax.experimental.pallas import tpu_sc as plsc

B = 8
LQ = 32
LK = 2048
D = 64
MAX_REL = 128
NROW = 2 * MAX_REL + 1
NPAIR = B * LQ
NW = 32
PAIRS_PER_W = NPAIR // NW
Q = 256
S = (PAIRS_PER_W * LK) // Q
STEPS_PER_PAIR = LK // Q
NSLOT = 2

_mesh = plsc.VectorSubcoreMesh(core_axis_name="c", subcore_axis_name="s")


@functools.partial(
    pl.kernel,
    mesh=_mesh,
    compiler_params=pltpu.CompilerParams(needs_layout_passes=False),
    out_type=jax.ShapeDtypeStruct((B, LQ, LK, D), jnp.float32),
    scratch_types=[
        pltpu.VMEM((NROW * D,), jnp.float32),
        pltpu.VMEM((LK,), jnp.int32),
        pltpu.VMEM((NPAIR + 16,), jnp.int32),
        pltpu.VMEM((NSLOT * Q, D), jnp.float32),
        pltpu.SemaphoreType.DMA,
        pltpu.SemaphoreType.DMA,
    ],
)
def _sc_gather(pq_hbm, pk_hbm, table_hbm, out_hbm, tab_v, pk_v, pq_v,
               rows_v, ssem0, ssem1):
    wid = lax.axis_index("s") * 2 + lax.axis_index("c")
    bq = wid // (LQ // PAIRS_PER_W)
    pltpu.sync_copy(table_hbm, tab_v)
    pltpu.sync_copy(pk_hbm.at[pl.ds(bq * LK, LK)], pk_v)
    pltpu.sync_copy(pq_hbm, pq_v)
    row_base = wid * (S * Q)
    lane = lax.iota(jnp.int32, 16)
    cvecs = [lane + c4 * 16 for c4 in range(D // 16)]
    _dnums = lax.GatherDimensionNumbers(
        offset_dims=(), collapsed_slice_dims=(0,), start_index_map=(0,)
    )

    def _splat_lane(vec, r):
        return lax.gather(
            vec, jnp.full((16, 1), r, jnp.int32), _dnums, (1,),
            mode=lax.GatherScatterMode.PROMISE_IN_BOUNDS,
        )

    def _splat_lane_dyn(vec, r):
        return lax.gather(
            vec, jnp.broadcast_to(r, (16, 1)).astype(jnp.int32), _dnums, (1,),
            mode=lax.GatherScatterMode.PROMISE_IN_BOUNDS,
        )

    def build_step(t, k):
        pq_scalar = pq_v[pl.ds(wid * PAIRS_PER_W + t // STEPS_PER_PAIR, 16)][0]
        pq_splat = jnp.full((16,), pq_scalar, jnp.int32)
        jbase = (t % STEPS_PER_PAIR) * Q

        def grp_body(g, carry):
            pk16 = pk_v[pl.ds(jbase + g * 16, 16)]
            flat = (jnp.clip(pk16 - pq_splat, -MAX_REL, MAX_REL) + MAX_REL) * D
            dst_g = k * Q + g * 16

            @plsc.parallel_loop(0, 16, 1, unroll=8)
            def _rows(r):
                rowbase = _splat_lane_dyn(flat, r)
                for c4 in range(D // 16):
                    vals = plsc.load_gather(tab_v, [rowbase + cvecs[c4]])
                    rows_v[dst_g + r, pl.ds(c4 * 16, 16)] = vals

            return carry

        lax.fori_loop(0, Q // 16, grp_body, 0)

    def fire_scatter(t, k, sem):
        iq = (wid % (LQ // PAIRS_PER_W)) * PAIRS_PER_W + t // STEPS_PER_PAIR
        jb = (t % STEPS_PER_PAIR) * Q
        pltpu.async_copy(
            rows_v.at[pl.ds(k * Q, Q)],
            out_hbm.at[bq, iq, pl.ds(jb, Q), :],
            sem,
        )

    def wait_scatter(k, sem):
        pltpu.make_async_copy(
            rows_v.at[pl.ds(k * Q, Q)],
            out_hbm.at[0, 0, pl.ds(0, Q), :],
            sem,
        ).wait()

    build_step(0, 0)
    fire_scatter(0, 0, ssem0)
    build_step(1, 1)
    fire_scatter(1, 1, ssem1)

    def outer(o, carry):
        s0 = o * 2
        wait_scatter(0, ssem0)
        build_step(s0, 0)
        fire_scatter(s0, 0, ssem0)
        wait_scatter(1, ssem1)
        build_step(s0 + 1, 1)
        fire_scatter(s0 + 1, 1, ssem1)
        return carry

    lax.fori_loop(1, S // 2, outer, 0)
    wait_scatter(0, ssem0)
    wait_scatter(1, ssem1)


def kernel(position_q, position_k, embeddings_table):
    pq = position_q.astype(jnp.int32).reshape(NPAIR)
    pq = jnp.pad(pq, (0, 16))
    pk = position_k.astype(jnp.int32).reshape(B * LK)
    tab = embeddings_table.reshape(NROW * D)
    return _sc_gather(pq, pk, tab)


# speedup vs baseline: 17.7584x; 1.0032x over previous
import functools

import jax
import jax.numpy as jnp
from jax import lax
from jax.experimental import pallas as pl
from jax.experimental.pallas import tpu as pltpu
from jax.experimental.pallas import tpu_sc as plsc

B = 8
LQ = 32
LK = 2048
D = 64
MAX_REL = 128
NROW = 2 * MAX_REL + 1
NPAIR = B * LQ
NW = 32
PAIRS_PER_W = NPAIR // NW
Q = 256
S = (PAIRS_PER_W * LK) // Q
STEPS_PER_PAIR = LK // Q
NSLOT = 2

_mesh = plsc.VectorSubcoreMesh(core_axis_name="c", subcore_axis_name="s")


@functools.partial(
    pl.kernel,
    mesh=_mesh,
    compiler_params=pltpu.CompilerParams(needs_layout_passes=False),
    out_type=jax.ShapeDtypeStruct((B, LQ, LK, D), jnp.float32),
    scratch_types=[
        pltpu.VMEM((NROW, D), jnp.float32),
        pltpu.VMEM((LK,), jnp.int32),
        pltpu.VMEM((NPAIR + 16,), jnp.int32),
        pltpu.VMEM((NSLOT * Q, D), jnp.float32),
        pltpu.SemaphoreType.DMA,
        pltpu.SemaphoreType.DMA,
    ],
)
def _sc_gather(pq_hbm, pk_hbm, table_hbm, out_hbm, tab_v, pk_v, pq_v,
               rows_v, ssem0, ssem1):
    wid = lax.axis_index("s") * 2 + lax.axis_index("c")
    bq = wid // (LQ // PAIRS_PER_W)
    h1 = pltpu.async_copy(table_hbm, tab_v, ssem0)
    h2 = pltpu.async_copy(pk_hbm.at[bq], pk_v, ssem0)
    h3 = pltpu.async_copy(pq_hbm, pq_v.at[pl.ds(0, NPAIR)], ssem0)
    h1.wait()
    h2.wait()
    h3.wait()
    lane = lax.iota(jnp.int32, 16)
    cvecs = [lane + c4 * 16 for c4 in range(D // 16)]
    _dnums = lax.GatherDimensionNumbers(
        offset_dims=(), collapsed_slice_dims=(0,), start_index_map=(0,)
    )

    def _splat_lane(vec, r):
        return lax.gather(
            vec, jnp.broadcast_to(r, (16, 1)).astype(jnp.int32), _dnums, (1,),
            mode=lax.GatherScatterMode.PROMISE_IN_BOUNDS,
        )

    def build_step(t, k):
        pq_scalar = pq_v[pl.ds(wid * PAIRS_PER_W + t // STEPS_PER_PAIR, 16)][0]
        pq_splat = jnp.full((16,), pq_scalar, jnp.int32)
        jbase = (t % STEPS_PER_PAIR) * Q

        def grp_body(g, carry):
            pk16 = pk_v[pl.ds(jbase + g * 16, 16)]
            rows16 = jnp.clip(pk16 - pq_splat, -MAX_REL, MAX_REL) + MAX_REL
            dst_g = k * Q + g * 16

            @plsc.parallel_loop(0, 16, 1, unroll=8)
            def _rows(r):
                rowsplat = _splat_lane(rows16, r)
                for c4 in range(D // 16):
                    vals = plsc.load_gather(tab_v, [rowsplat, cvecs[c4]])
                    rows_v[dst_g + r, pl.ds(c4 * 16, 16)] = vals

            return carry

        lax.fori_loop(0, Q // 16, grp_body, 0)

    def fire_scatter(t, k, sem):
        iq = (wid % (LQ // PAIRS_PER_W)) * PAIRS_PER_W + t // STEPS_PER_PAIR
        jb = (t % STEPS_PER_PAIR) * Q
        pltpu.async_copy(
            rows_v.at[pl.ds(k * Q, Q)],
            out_hbm.at[bq, iq, pl.ds(jb, Q), :],
            sem,
        )

    def wait_scatter(k, sem):
        pltpu.make_async_copy(
            rows_v.at[pl.ds(k * Q, Q)],
            out_hbm.at[0, 0, pl.ds(0, Q), :],
            sem,
        ).wait()

    build_step(0, 0)
    fire_scatter(0, 0, ssem0)
    build_step(1, 1)
    fire_scatter(1, 1, ssem1)

    def outer(o, carry):
        s0 = o * 2
        wait_scatter(0, ssem0)
        build_step(s0, 0)
        fire_scatter(s0, 0, ssem0)
        wait_scatter(1, ssem1)
        build_step(s0 + 1, 1)
        fire_scatter(s0 + 1, 1, ssem1)
        return carry

    lax.fori_loop(1, S // 2, outer, 0)
    wait_scatter(0, ssem0)
    wait_scatter(1, ssem1)


def kernel(position_q, position_k, embeddings_table):
    pq = position_q.astype(jnp.int32).reshape(NPAIR)
    pk = position_k.astype(jnp.int32)
    return _sc_gather(pq, pk, embeddings_table)
